# baseline (speedup 1.0000x reference)
import functools

import jax
import jax.numpy as jnp
from jax import lax
from jax.experimental import pallas as pl
from jax.experimental.pallas import tpu as pltpu
from jax.experimental.pallas import tpu_sc as plsc

B, N, D = 4096, 128, 64
NW = 32
B_SC = 2048
BPW = B_SC // NW
G = 8
L = 16
U = D // L
SIGN_BIT = 0x80000000


def _sc_loss(pred, gt, am):
    mesh = plsc.VectorSubcoreMesh(core_axis_name="c", subcore_axis_name="s")

    @functools.partial(
        pl.kernel,
        mesh=mesh,
        out_type=jax.ShapeDtypeStruct((NW, L), jnp.float32),
        compiler_params=pltpu.CompilerParams(needs_layout_passes=False),
        scratch_types=[
            pltpu.VMEM((N, D), jnp.float32),
            pltpu.VMEM((N, D), jnp.float32),
            pltpu.VMEM((N, D), jnp.float32),
            pltpu.VMEM((N, D), jnp.float32),
            pltpu.VMEM((N, D), jnp.int32),
            pltpu.VMEM((N, D), jnp.int32),
            pltpu.VMEM((U * L,), jnp.float32),
            pltpu.VMEM((L,), jnp.float32),
            pltpu.SemaphoreType.DMA,
            pltpu.SemaphoreType.DMA,
        ],
    )
    def k(pred_hbm, gt_hbm, am_hbm, out_hbm, pred_v0, pred_v1, gt_v0, gt_v1,
          am_v0, am_v1, accb, acc_v, sem0, sem1):
        wid = lax.axis_index("s") * 2 + lax.axis_index("c")
        base_b = wid * BPW
        iota = lax.iota(jnp.int32, L)
        dvecs = [u * L + iota for u in range(U)]
        slots = ((pred_v0, gt_v0, am_v0), (pred_v1, gt_v1, am_v1))

        def start(i, slot, sem):
            b = base_b + i
            pv, gv, av = slots[slot]
            pltpu.async_copy(pred_hbm.at[b], pv, sem)
            pltpu.async_copy(gt_hbm.at[b], gv, sem)
            pltpu.async_copy(am_hbm.at[b], av, sem)

        def drain(i, slot, sem):
            b = base_b + i
            pv, gv, av = slots[slot]
            pltpu.make_async_copy(pred_hbm.at[b], pv, sem).wait()
            pltpu.make_async_copy(gt_hbm.at[b], gv, sem).wait()
            pltpu.make_async_copy(am_hbm.at[b], av, sem).wait()

        def compute(slot):
            pv, gv, av = slots[slot]

            @plsc.parallel_loop(0, N)
            def row(n):
                for u in range(U):
                    s = pl.ds(u * L, L)
                    e = pv[n, s] - gv[n, s]
                    a = av[n, s]
                    u_g = (plsc.load_gather(pv, [a, dvecs[u]])
                           - plsc.load_gather(gv, [a, dvecs[u]]))
                    t = e * jnp.abs(e)
                    r = plsc.bitcast(
                        plsc.bitcast(t, jnp.uint32)
                        ^ (plsc.bitcast(u_g, jnp.uint32)
                           & jnp.uint32(SIGN_BIT)),
                        jnp.float32)
                    plsc.addupdate(accb.at[pl.ds(u * L, L)], r)

        for u in range(U):
            accb[pl.ds(u * L, L)] = jnp.zeros((L,), jnp.float32)
        start(0, 0, sem0)

        def outer(j, carry):
            i0 = 2 * j
            start(i0 + 1, 1, sem1)
            drain(i0, 0, sem0)
            compute(0)
            start((i0 + 2) % BPW, 0, sem0)
            drain(i0 + 1, 1, sem1)
            compute(1)
            return carry

        lax.fori_loop(0, BPW // 2, outer, jnp.int32(0))
        drain(0, 0, sem0)
        acc_v[...] = (accb[pl.ds(0, L)] + accb[pl.ds(L, L)]
                      + accb[pl.ds(2 * L, L)] + accb[pl.ds(3 * L, L)])
        pltpu.sync_copy(acc_v, out_hbm.at[wid])

    return k(pred, gt, am)


def _tc_loss(pred, gt, am):
    grid = ((B - B_SC) // G,)

    def body(p_ref, g_ref, a_ref, out_ref):
        i = pl.program_id(0)
        e = p_ref[...] - g_ref[...]
        t = e * jnp.abs(e)
        a = a_ref[...]
        neg = (e < 0).astype(jnp.int32)
        w = neg.reshape(G, 4, 32, D)
        shifts = lax.broadcasted_iota(jnp.int32, (G, 4, 32, D), 2)
        packed = jnp.sum(w << shifts, axis=2)
        j = a >> 5
        word = jnp.where(j < 2,
                         jnp.where(j == 0, packed[:, 0:1, :],
                                   packed[:, 1:2, :]),
                         jnp.where(j == 2, packed[:, 2:3, :],
                                   packed[:, 3:4, :]))
        bit = (word >> (a & 31)) & 1
        r = lax.bitcast_convert_type(t, jnp.int32) ^ (bit << 31)
        r = lax.bitcast_convert_type(r, jnp.float32)
        s = jnp.sum(r, axis=2)

        @pl.when(i == 0)
        def _():
            out_ref[...] = jnp.zeros_like(out_ref)
        out_ref[...] += s

    return pl.pallas_call(
        body,
        grid=grid,
        in_specs=[
            pl.BlockSpec((G, N, D), lambda i: (B_SC // G + i, 0, 0)),
            pl.BlockSpec((G, N, D), lambda i: (B_SC // G + i, 0, 0)),
            pl.BlockSpec((G, N, D), lambda i: (B_SC // G + i, 0, 0)),
        ],
        out_specs=pl.BlockSpec((G, N), lambda i: (0, 0)),
        out_shape=jax.ShapeDtypeStruct((G, N), jnp.float32),
        compiler_params=pltpu.CompilerParams(
            dimension_semantics=("arbitrary",)),
    )(pred, gt, am)


def kernel(predictions, ground_truth, anchor_masks):
    am = anchor_masks.astype(jnp.int32)
    sc = _sc_loss(predictions, ground_truth, am)
    tc = _tc_loss(predictions, ground_truth, am)
    return (jnp.sum(sc) + jnp.sum(tc)) / jnp.float32(B * N * D)
